# baseline (speedup 1.0000x reference)
import functools

import jax
import jax.numpy as jnp
from jax import lax
from jax.experimental import pallas as pl
from jax.experimental.pallas import tpu as pltpu
from jax.experimental.pallas import tpu_sc as plsc

VOCAB, WIDTH, CTX, BATCH = 49408, 512, 77, 1024
LANES = 16
NUM_CORES = 2
NUM_SUBCORES = 16


CTX_PAD = 80


def kernel(input_ts, tokenembd, pstnembd):
    nw = NUM_CORES * NUM_SUBCORES
    rows_per_w = BATCH // nw

    mesh = plsc.VectorSubcoreMesh(core_axis_name="c", subcore_axis_name="s")

    @functools.partial(
        pl.kernel,
        out_type=jax.ShapeDtypeStruct((BATCH, CTX, WIDTH), jnp.float32),
        mesh=mesh,
        scratch_types=[
            pltpu.VMEM((BATCH // (NUM_CORES * NUM_SUBCORES), CTX_PAD), jnp.int32),
            pltpu.VMEM((CTX, WIDTH), jnp.float32),
            pltpu.VMEM((CTX_PAD, WIDTH), jnp.float32),
            pltpu.VMEM((CTX_PAD, WIDTH), jnp.float32),
            pltpu.SemaphoreType.DMA,
            pltpu.SemaphoreType.DMA,
            pltpu.SemaphoreType.DMA,
            pltpu.SemaphoreType.DMA,
        ],
        compiler_params=pltpu.CompilerParams(use_tc_tiling_on_sc=False),
    )
    def launch(idx_hbm, tok_hbm, pos_hbm, out_hbm,
               idx_all, pos_v, buf0, buf1, g0, g1, s0, s1):
        wid = lax.axis_index("s") * NUM_CORES + lax.axis_index("c")
        base = wid * rows_per_w
        pltpu.sync_copy(pos_hbm, pos_v)
        pltpu.sync_copy(idx_hbm.at[pl.ds(base, rows_per_w)], idx_all)

        bufs, gsems, ssems = (buf0, buf1), (g0, g1), (s0, s1)
        gh = [None] * rows_per_w
        sh = [None] * rows_per_w
        gh[0] = pltpu.async_copy(tok_hbm.at[idx_all.at[0]], bufs[0], gsems[0])
        for b in range(rows_per_w):
            slot, nslot = b % 2, (b + 1) % 2
            if b + 1 < rows_per_w:
                if b >= 1:
                    sh[b - 1].wait()
                gh[b + 1] = pltpu.async_copy(
                    tok_hbm.at[idx_all.at[b + 1]], bufs[nslot], gsems[nslot])
            gh[b].wait()
            buf = bufs[slot]

            def add_row(r, c2, buf=buf):
                for g in range(WIDTH // LANES):
                    sl = pl.ds(g * LANES, LANES)
                    buf[r, sl] = buf[r, sl] + pos_v[r, sl]
                return c2

            lax.fori_loop(0, CTX, add_row, 0)
            sh[b] = pltpu.async_copy(
                buf.at[pl.ds(0, CTX)], out_hbm.at[base + b], ssems[slot])
        sh[rows_per_w - 2].wait()
        sh[rows_per_w - 1].wait()

    idx_pad = jnp.pad(input_ts.astype(jnp.int32), ((0, 0), (0, CTX_PAD - CTX)))
    return launch(idx_pad, tokenembd, pstnembd)


# speedup vs baseline: 1.3562x; 1.3562x over previous
import functools

import jax
import jax.numpy as jnp
from jax import lax
from jax.experimental import pallas as pl
from jax.experimental.pallas import tpu as pltpu
from jax.experimental.pallas import tpu_sc as plsc

VOCAB, WIDTH, CTX, BATCH = 49408, 512, 77, 1024
CTXP = 80
LANES = 16
NUM_CORES = 2
NUM_SUBCORES = 16
NW = NUM_CORES * NUM_SUBCORES
ROWS_PER_W = BATCH // NW


def kernel(input_ts, tokenembd, pstnembd):
    mesh = plsc.VectorSubcoreMesh(core_axis_name="c", subcore_axis_name="s")

    @functools.partial(
        pl.kernel,
        out_type=jax.ShapeDtypeStruct((BATCH, CTX, WIDTH), jnp.float32),
        mesh=mesh,
        scratch_types=[
            pltpu.VMEM((ROWS_PER_W, 1, CTXP), jnp.int32),
            pltpu.VMEM((CTX, WIDTH), jnp.float32),
            pltpu.VMEM((CTXP, WIDTH), jnp.float32),
            pltpu.VMEM((CTX, WIDTH), jnp.float32),
            pltpu.SemaphoreType.DMA,
        ],
    )
    def launch(idx_hbm, tok_hbm, pos_hbm, out_hbm,
               idx_all, pos_v, buf, sbuf, gsem):
        wid = lax.axis_index("s") * NUM_CORES + lax.axis_index("c")
        base = wid * ROWS_PER_W
        pltpu.sync_copy(idx_hbm.at[wid], idx_all)
        pltpu.sync_copy(pos_hbm, pos_v)

        def add_groups(g, c2):
            sl = pl.ds(g * LANES, LANES)
            for rr in range(CTX):
                sbuf[rr, sl] = buf[rr, sl] + pos_v[rr, sl]
            return c2

        @pl.loop(0, ROWS_PER_W)
        def row_loop(r):
            pltpu.async_copy(tok_hbm.at[idx_all.at[r, 0]], buf, gsem).wait()
            lax.fori_loop(0, WIDTH // LANES, add_groups, 0)
            pltpu.sync_copy(sbuf, out_hbm.at[base + r])

    idx = jnp.pad(input_ts.astype(jnp.int32), ((0, 0), (0, CTXP - CTX)))
    return launch(idx.reshape(NW, ROWS_PER_W, 1, CTXP), tokenembd, pstnembd)


# speedup vs baseline: 1.4234x; 1.0495x over previous
import functools

import jax
import jax.numpy as jnp
from jax import lax
from jax.experimental import pallas as pl
from jax.experimental.pallas import tpu as pltpu
from jax.experimental.pallas import tpu_sc as plsc

VOCAB, WIDTH, CTX, BATCH = 49408, 512, 77, 1024
CTXP = 80
LANES = 16
NUM_CORES = 2
NUM_SUBCORES = 16
NW = NUM_CORES * NUM_SUBCORES
ROWS_PER_W = BATCH // NW


def kernel(input_ts, tokenembd, pstnembd):
    mesh = plsc.VectorSubcoreMesh(core_axis_name="c", subcore_axis_name="s")

    @functools.partial(
        pl.kernel,
        out_type=jax.ShapeDtypeStruct((BATCH, CTX, WIDTH), jnp.float32),
        mesh=mesh,
        scratch_types=[
            pltpu.VMEM((ROWS_PER_W, 1, CTXP), jnp.int32),
            pltpu.VMEM((CTX, WIDTH), jnp.float32),
            pltpu.VMEM((CTXP, WIDTH), jnp.float32),
            pltpu.VMEM((CTX, WIDTH), jnp.float32),
            pltpu.SemaphoreType.DMA,
        ],
    )
    def launch(idx_hbm, tok_hbm, pos_hbm, out_hbm,
               idx_all, pos_v, buf, sbuf, gsem):
        wid = lax.axis_index("s") * NUM_CORES + lax.axis_index("c")
        base = wid * ROWS_PER_W
        pltpu.sync_copy(idx_hbm.at[wid], idx_all)
        pltpu.sync_copy(pos_hbm, pos_v)

        def add_groups(g, c2):
            sl = pl.ds(g * LANES, LANES)
            for rr in range(CTX):
                sbuf[rr, sl] = buf[rr, sl] + pos_v[rr, sl]
            return c2

        pltpu.async_copy(tok_hbm.at[idx_all.at[0, 0]], buf, gsem)

        @pl.loop(0, ROWS_PER_W)
        def row_loop(r):
            pltpu.make_async_copy(tok_hbm.at[idx_all.at[0, 0]], buf,
                                  gsem).wait()
            lax.fori_loop(0, WIDTH // LANES, add_groups, 0)

            @pl.when(r + 1 < ROWS_PER_W)
            def _prefetch():
                pltpu.async_copy(tok_hbm.at[idx_all.at[r + 1, 0]], buf, gsem)

            pltpu.sync_copy(sbuf, out_hbm.at[base + r])

    idx = jnp.pad(input_ts.astype(jnp.int32), ((0, 0), (0, CTXP - CTX)))
    return launch(idx.reshape(NW, ROWS_PER_W, 1, CTXP), tokenembd, pstnembd)
